# baseline (speedup 1.0000x reference)
import functools

import jax
import jax.numpy as jnp
import numpy as np
from jax import lax
from jax.experimental import pallas as pl
from jax.experimental.pallas import tpu as pltpu
from jax.experimental.pallas import tpu_sc as plsc

RESOLUTION = 1000
D = 64
B = 16384
S = 26
N = B * S

NC = 2
NS = 16
NW = NC * NS
PER_W = N // NW
LANES = 16

BLOCK = 512
NB = PER_W // BLOCK
NPAIR = NB // 2
SUB = 128
NSUB = BLOCK // SUB

CLIP_LO = np.float32(1.0 / RESOLUTION)
SCALE = np.float32(RESOLUTION)
MAGIC = np.float32(8388608.0)


def _body(x_hbm, pe_hbm, out_hbm, x_v, idx_v, buf0, buf1, gsem0, gsem1,
          wsem0, wsem1):
    wid = lax.axis_index("s") * NC + lax.axis_index("c")
    base = wid * PER_W

    pltpu.sync_copy(x_hbm.at[pl.ds(base, PER_W)], x_v)

    def idx_body(i, carry):
        off = i * LANES
        v = x_v[pl.ds(off, LANES)]
        t = jnp.maximum(v, CLIP_LO) * SCALE
        r = (t + MAGIC) - MAGIC
        idx_v[pl.ds(off, LANES)] = r.astype(jnp.int32) - 1
        return carry

    lax.fori_loop(0, PER_W // LANES, idx_body, 0)

    def fire_gather(k, buf, sem):
        return [
            pltpu.async_copy(
                pe_hbm.at[idx_v.at[pl.ds(k * BLOCK + j * SUB, SUB)]],
                buf.at[pl.ds(j * SUB, SUB)],
                sem,
            )
            for j in range(NSUB)
        ]

    def wait_gather(buf, sem):
        for j in range(NSUB):
            pltpu.make_async_copy(
                pe_hbm.at[idx_v.at[pl.ds(j * SUB, SUB)]],
                buf.at[pl.ds(j * SUB, SUB)],
                sem,
            ).wait()

    def fire_write(k, buf, sem):
        return pltpu.async_copy(buf, out_hbm.at[pl.ds(base + k * BLOCK, BLOCK)], sem)

    def wait_write(buf, sem):
        pltpu.make_async_copy(buf, out_hbm.at[pl.ds(base, BLOCK)], sem).wait()

    def pair_body(g, fire_next_gather, first, last):
        a = 2 * g
        b = 2 * g + 1
        wait_gather(buf0, gsem0)
        fire_write(a, buf0, wsem0)
        if not first:
            wait_write(buf1, wsem1)
        fire_gather(b, buf1, gsem1)
        wait_gather(buf1, gsem1)
        fire_write(b, buf1, wsem1)
        wait_write(buf0, wsem0)
        if fire_next_gather:
            fire_gather(a + 2, buf0, gsem0)

    fire_gather(0, buf0, gsem0)

    def loop_body(g, carry):
        pair_body(g, fire_next_gather=True, first=False, last=False)
        return carry

    pair_body(0, fire_next_gather=True, first=True, last=False)
    lax.fori_loop(1, NPAIR - 1, loop_body, 0)
    pair_body(NPAIR - 1, fire_next_gather=False, first=False, last=True)
    wait_write(buf1, wsem1)


@jax.jit
def _encode(x_flat, pe):
    mesh = plsc.VectorSubcoreMesh(
        core_axis_name="c", subcore_axis_name="s", num_cores=NC, num_subcores=NS
    )
    return pl.kernel(
        _body,
        out_type=jax.ShapeDtypeStruct((N, D), jnp.float32),
        mesh=mesh,
        scratch_types=[
            pltpu.VMEM((PER_W,), jnp.float32),
            pltpu.VMEM((PER_W,), jnp.int32),
            pltpu.VMEM((BLOCK, D), jnp.float32),
            pltpu.VMEM((BLOCK, D), jnp.float32),
            pltpu.SemaphoreType.DMA,
            pltpu.SemaphoreType.DMA,
            pltpu.SemaphoreType.DMA,
            pltpu.SemaphoreType.DMA,
        ],
        compiler_params=pltpu.CompilerParams(use_tc_tiling_on_sc=False),
    )(x_flat, pe)


def kernel(x, pe):
    out = _encode(x.reshape(N), pe)
    return out.reshape(B, S, D)


# speedup vs baseline: 1.3308x; 1.3308x over previous
import functools

import jax
import jax.numpy as jnp
import numpy as np
from jax import lax
from jax.experimental import pallas as pl
from jax.experimental.pallas import tpu as pltpu
from jax.experimental.pallas import tpu_sc as plsc

RESOLUTION = 1000
D = 64
B = 16384
S = 26
N = B * S

NC = 2
NS = 16
NW = NC * NS
BPW = B // NW
PER_W = N // NW
LANES = 16

DT = D // 8
BT = B // 128
BT_W = BT // NW
NG = 128 // LANES

CLIP_LO = np.float32(1.0 / RESOLUTION)
SCALE = np.float32(RESOLUTION)
MAGIC = np.float32(8388608.0)


def _body(x_hbm, peT_hbm, out_hbm, x_v, idx_v, pe_v, buf0, buf1,
          wsem0, wsem1):
    wid = lax.axis_index("s") * NC + lax.axis_index("c")
    base = wid * PER_W

    pltpu.sync_copy(peT_hbm, pe_v)
    pltpu.sync_copy(x_hbm.at[pl.ds(base, PER_W)], x_v)

    def idx_body(i, carry):
        off = i * LANES
        v = x_v[pl.ds(off, LANES)]
        t = jnp.maximum(v, CLIP_LO) * SCALE
        r = (t + MAGIC) - MAGIC
        idx_v[pl.ds(off, LANES)] = r.astype(jnp.int32) - 1
        return carry

    lax.fori_loop(0, PER_W // LANES, idx_body, 0)

    lane_s = lax.iota(jnp.int32, LANES) * S

    bufs = (buf0, buf1)
    wsems = (wsem0, wsem1)

    def wait_write(p):
        pltpu.make_async_copy(bufs[p], out_hbm.at[0, 0, 0], wsems[p]).wait()

    def s_tile(bt_l, s, first):
        bt = wid * BT_W + bt_l
        ivecs = [
            plsc.load_gather(
                idx_v, [lane_s + ((bt_l * 128 + g * LANES) * S + s)]
            )
            for g in range(NG)
        ]
        for dt in range(DT):
            p = dt % 2
            buf = bufs[p]
            if not (first and dt < 2):
                wait_write(p)
            for dl in range(8):
                rowoff = (dt * 8 + dl) * RESOLUTION
                for g in range(NG):
                    v = plsc.load_gather(pe_v, [ivecs[g] + rowoff])
                    buf[dl, pl.ds(g * LANES, LANES)] = v
            pltpu.async_copy(buf, out_hbm.at[s, dt, bt], wsems[p])

    s_tile(0, 0, first=True)

    def loop_body(k, carry):
        s_tile(k // S, k % S, first=False)
        return carry

    lax.fori_loop(1, BT_W * S, loop_body, 0)
    wait_write(0)
    wait_write(1)


@jax.jit
def _encode(x_flat, peT_flat):
    mesh = plsc.VectorSubcoreMesh(
        core_axis_name="c", subcore_axis_name="s", num_cores=NC, num_subcores=NS
    )
    return pl.kernel(
        _body,
        out_type=jax.ShapeDtypeStruct((S, DT, BT, 8, 128), jnp.float32),
        mesh=mesh,
        scratch_types=[
            pltpu.VMEM((PER_W,), jnp.float32),
            pltpu.VMEM((PER_W,), jnp.int32),
            pltpu.VMEM((RESOLUTION * D,), jnp.float32),
            pltpu.VMEM((8, 128), jnp.float32),
            pltpu.VMEM((8, 128), jnp.float32),
            pltpu.SemaphoreType.DMA,
            pltpu.SemaphoreType.DMA,
        ],
        compiler_params=pltpu.CompilerParams(
            use_tc_tiling_on_sc=False, needs_layout_passes=False
        ),
    )(x_flat, peT_flat)


def kernel(x, pe):
    peT_flat = pe.T.reshape(RESOLUTION * D)
    out5d = _encode(x.reshape(N), peT_flat)
    return out5d.transpose(2, 4, 0, 1, 3).reshape(B, S, D)


# speedup vs baseline: 1.7321x; 1.3015x over previous
import functools

import jax
import jax.numpy as jnp
import numpy as np
from jax import lax
from jax.experimental import pallas as pl
from jax.experimental.pallas import tpu as pltpu
from jax.experimental.pallas import tpu_sc as plsc

RESOLUTION = 1000
D = 64
B = 16384
S = 26
N = B * S

NC = 2
NS = 16
NW = NC * NS
BPW = B // NW
PER_W = N // NW
LANES = 16

DT = D // 8
BT = B // 128
BT_W = BT // NW
NG = 128 // LANES

CLIP_LO = np.float32(1.0 / RESOLUTION)
SCALE = np.float32(RESOLUTION)
MAGIC = np.float32(8388608.0)


def _body(x_hbm, peT_hbm, out_hbm, x_v, idx_v, pe_v, buf0, buf1,
          wsem0, wsem1):
    wid = lax.axis_index("s") * NC + lax.axis_index("c")
    base = wid * PER_W

    pltpu.sync_copy(peT_hbm, pe_v)
    pltpu.sync_copy(x_hbm.at[pl.ds(base, PER_W)], x_v)

    def idx_body(i, carry):
        off = i * LANES
        v = x_v[pl.ds(off, LANES)]
        t = jnp.maximum(v, CLIP_LO) * SCALE
        r = (t + MAGIC) - MAGIC
        idx_v[pl.ds(off, LANES)] = r.astype(jnp.int32) - 1
        return carry

    lax.fori_loop(0, PER_W // LANES, idx_body, 0)

    lane_s = lax.iota(jnp.int32, LANES) * S

    def wait_write(buf, wsem):
        pltpu.make_async_copy(buf, out_hbm.at[0, :, 0], wsem).wait()

    def s_tile(t, buf, wsem):
        bt_l = t // S
        s = t % S
        bt = wid * BT_W + bt_l
        ivecs = [
            plsc.load_gather(
                idx_v, [lane_s + ((bt_l * 128 + g * LANES) * S + s)]
            )
            for g in range(NG)
        ]

        def dt_body(dt, carry):
            for dl in range(8):
                rowoff = (dt * 8 + dl) * RESOLUTION
                for g in range(NG):
                    v = plsc.load_gather(pe_v, [ivecs[g] + rowoff])
                    buf[dt, dl, pl.ds(g * LANES, LANES)] = v
            return carry

        lax.fori_loop(0, DT, dt_body, 0)
        pltpu.async_copy(buf, out_hbm.at[s, :, bt], wsem)

    NT = BT_W * S
    s_tile(0, buf0, wsem0)
    s_tile(1, buf1, wsem1)

    def loop_body(g, carry):
        wait_write(buf0, wsem0)
        s_tile(2 * g, buf0, wsem0)
        wait_write(buf1, wsem1)
        s_tile(2 * g + 1, buf1, wsem1)
        return carry

    lax.fori_loop(1, NT // 2, loop_body, 0)
    wait_write(buf0, wsem0)
    wait_write(buf1, wsem1)


@jax.jit
def _encode(x_flat, peT_flat):
    mesh = plsc.VectorSubcoreMesh(
        core_axis_name="c", subcore_axis_name="s", num_cores=NC, num_subcores=NS
    )
    return pl.kernel(
        _body,
        out_type=jax.ShapeDtypeStruct((S, DT, BT, 8, 128), jnp.float32),
        mesh=mesh,
        scratch_types=[
            pltpu.VMEM((PER_W,), jnp.float32),
            pltpu.VMEM((PER_W,), jnp.int32),
            pltpu.VMEM((RESOLUTION * D,), jnp.float32),
            pltpu.VMEM((DT, 8, 128), jnp.float32),
            pltpu.VMEM((DT, 8, 128), jnp.float32),
            pltpu.SemaphoreType.DMA,
            pltpu.SemaphoreType.DMA,
        ],
        compiler_params=pltpu.CompilerParams(
            use_tc_tiling_on_sc=False, needs_layout_passes=False
        ),
    )(x_flat, peT_flat)


def kernel(x, pe):
    peT_flat = pe.T.reshape(RESOLUTION * D)
    out5d = _encode(x.reshape(N), peT_flat)
    return out5d.transpose(2, 4, 0, 1, 3).reshape(B, S, D)
